# Initial kernel scaffold; baseline (speedup 1.0000x reference)
import jax
import jax.numpy as jnp
from jax.experimental import pallas as pl


def kernel(queries, keys, values):
    raise NotImplementedError("write your pallas kernel here")


# speedup vs baseline: 5.3408x; 5.3408x over previous
import math

import jax
import jax.numpy as jnp
import numpy as np
from jax.experimental import pallas as pl
from jax.experimental.pallas import tpu as pltpu

_B, _L, _H, _D = 1, 2048, 16, 64
_SAMPLE_K = min(5 * math.ceil(math.log(_L)), _L)
_N_TOP = min(5 * math.ceil(math.log(_L)), _L)
_U = _D
_QT = 1024
_NQ = _L // _QT
_NEG = -1e30


def _threefry2x32(k1, k2, x0, x1):
    rot = ((13, 15, 26, 6), (17, 29, 16, 24))
    ks = (np.uint32(k1), np.uint32(k2),
          np.uint32(k1) ^ np.uint32(k2) ^ np.uint32(0x1BD11BDA))
    x = [x0.astype(np.uint32) + ks[0], x1.astype(np.uint32) + ks[1]]
    for i in range(5):
        for r in rot[i % 2]:
            a = x[0] + x[1]
            b = (x[1] << np.uint32(r)) | (x[1] >> np.uint32(32 - r))
            x = [a, a ^ b]
        x[0] = x[0] + ks[(i + 1) % 3]
        x[1] = x[1] + ks[(i + 2) % 3] + np.uint32(i + 1)
    return x


def _sample_idx() -> np.ndarray:
    n = _L * _SAMPLE_K
    with np.errstate(over="ignore"):
        sk1, sk2 = _threefry2x32(0, 42, np.zeros(2, np.uint32),
                                 np.arange(2, dtype=np.uint32))
        zeros = np.zeros(n, np.uint32)
        iota = np.arange(n, dtype=np.uint32)
        hi1, hi2 = _threefry2x32(sk1[0], sk2[0], zeros, iota)
        lo1, lo2 = _threefry2x32(sk1[1], sk2[1], zeros, iota)
        higher = hi1 ^ hi2
        lower = lo1 ^ lo2
        span = np.uint32(_L)
        mult = np.uint32((2 ** 16 % _L) * (2 ** 16 % _L) % _L)
        off = ((higher % span) * mult + lower % span) % span
    return off.astype(np.int32).reshape(_L, _SAMPLE_K)


def _sample_cnt() -> np.ndarray:
    idx = _sample_idx()
    cnt = np.zeros((_L, _L), np.float32)
    np.add.at(cnt, (np.arange(_L)[:, None], idx), 1.0)
    return cnt


_CNT = _sample_cnt()


def _saliency_kernel(q_ref, k_ref, cnt_ref, m_ref):
    q = q_ref[0]
    k = k_ref[0]
    s = jax.lax.dot_general(
        q, k, (((1,), (1,)), ((), ())),
        preferred_element_type=jnp.float32,
        precision=jax.lax.Precision.DEFAULT)
    c = cnt_ref[...]
    smax = jnp.max(jnp.where(c > 0.0, s, _NEG), axis=1)
    ssum = jnp.sum(s * c, axis=1)
    m_ref[0, 0, :] = smax - ssum / float(_L)


def _context_kernel(m_ref, q_ref, k0_ref, v_ref, updfix_ref, gidx_ref, oh_ref):
    iota = jax.lax.broadcasted_iota(jnp.int32, (_H, _L), 1)
    cur = m_ref[...]
    for t in range(_N_TOP):
        mx = jnp.max(cur, axis=1, keepdims=True)
        hit = cur == mx
        pos = jnp.min(jnp.where(hit, iota, _L), axis=1, keepdims=True)
        sel = iota == pos
        oh_ref[:, t, :] = jnp.where(sel, 1.0, 0.0)
        cur = jnp.where(sel, _NEG, cur)
    j_b = jax.lax.broadcasted_iota(jnp.int32, (_U, _U), 1)
    for h in range(_H):
        oh = oh_ref[h]
        qr = jax.lax.dot_general(
            oh, q_ref[h], (((1,), (0,)), ((), ())),
            preferred_element_type=jnp.float32,
            precision=jax.lax.Precision.HIGHEST)
        qmax = jnp.max(qr, axis=1, keepdims=True)
        e = jnp.exp(qr - qmax)
        attn = e / jnp.sum(e, axis=1, keepdims=True)
        upd = jax.lax.dot_general(
            attn, v_ref[h], (((1,), (0,)), ((), ())),
            preferred_element_type=jnp.float32,
            precision=jax.lax.Precision.DEFAULT)
        upd64 = jnp.concatenate(
            [upd, jnp.zeros((_U - _N_TOP, _D), jnp.float32)], axis=0)
        idx2 = jnp.clip(k0_ref[h].astype(jnp.int32), 0, _L - 1)
        same = idx2[None, :] == idx2[:, None]
        jwin = jnp.max(jnp.where(same, j_b, -1), axis=1, keepdims=True)
        perm = jnp.where(j_b == jwin, 1.0, 0.0)
        winner_rows = jax.lax.dot_general(
            perm, upd64, (((1,), (0,)), ((), ())),
            preferred_element_type=jnp.float32,
            precision=jax.lax.Precision.HIGHEST)
        updfix_ref[h] = jnp.concatenate(
            [winner_rows, jnp.zeros((_U, _PD - _D), jnp.float32)], axis=1)
        gidx_ref[h] = idx2 + h * _L


_ZR = 256
_PD = 128


def _sc_scatter_body(upd_hbm, gidx_hbm, zeros_hbm, out_hbm,
                     upd_v, gidx_v, zero_v, zsem, ssem):
    wid = jax.lax.axis_index("s") * 2 + jax.lax.axis_index("c")

    @pl.when(wid < _H)
    def _():
        h = wid
        pltpu.sync_copy(zeros_hbm, zero_v)
        pltpu.sync_copy(upd_hbm.at[h], upd_v)
        pltpu.sync_copy(gidx_hbm.at[h], gidx_v)
        copies = [
            pltpu.async_copy(
                zero_v, out_hbm.at[pl.ds(h * _L + i * _ZR, _ZR)], zsem)
            for i in range(_L // _ZR)
        ]
        for cp in copies:
            cp.wait()
        pltpu.async_copy(upd_v, out_hbm.at[gidx_v], ssem).wait()


def _make_sc_scatter():
    from jax.experimental.pallas import tpu_sc as plsc
    mesh = plsc.VectorSubcoreMesh(core_axis_name="c", subcore_axis_name="s")
    return pl.kernel(
        _sc_scatter_body,
        mesh=mesh,
        out_type=jax.ShapeDtypeStruct((_H * _L, _PD), jnp.float32),
        scratch_types=[
            pltpu.VMEM((_U, _PD), jnp.float32),
            pltpu.VMEM((_U,), jnp.int32),
            pltpu.VMEM((_ZR, _PD), jnp.float32),
            pltpu.SemaphoreType.DMA,
            pltpu.SemaphoreType.DMA,
        ],
    )


_SC_SCATTER_CACHE = []


def _sc_scatter(updfix, gidx, zeros):
    if not _SC_SCATTER_CACHE:
        _SC_SCATTER_CACHE.append(_make_sc_scatter())
    return _SC_SCATTER_CACHE[0](updfix, gidx, zeros)


def kernel(queries, keys, values):
    q = jnp.swapaxes(queries, 1, 2)[0]
    k = jnp.swapaxes(keys, 1, 2)[0]
    v = jnp.swapaxes(values, 1, 2)[0]
    cnt = jnp.asarray(_CNT)
    m = pl.pallas_call(
        _saliency_kernel,
        grid=(_NQ, _H),
        in_specs=[
            pl.BlockSpec((1, _QT, _D), lambda nq, h: (h, nq, 0)),
            pl.BlockSpec((1, _L, _D), lambda nq, h: (h, 0, 0)),
            pl.BlockSpec((_QT, _L), lambda nq, h: (nq, 0)),
        ],
        out_specs=pl.BlockSpec((1, 1, _QT), lambda nq, h: (h * _NQ + nq, 0, 0)),
        out_shape=jax.ShapeDtypeStruct((_H * _NQ, 1, _QT), jnp.float32),
    )(q, k, cnt)
    m = m.reshape(_H, _L)
    k0 = keys[0, 0]
    v64 = v[:, :_U, :]
    updfix, gidx = pl.pallas_call(
        _context_kernel,
        grid=(1,),
        in_specs=[
            pl.BlockSpec((_H, _L), lambda i: (0, 0)),
            pl.BlockSpec((_H, _L, _D), lambda i: (0, 0, 0)),
            pl.BlockSpec((_H, _D), lambda i: (0, 0)),
            pl.BlockSpec((_H, _U, _D), lambda i: (0, 0, 0)),
        ],
        out_specs=[
            pl.BlockSpec((_H, _U, _PD), lambda i: (0, 0, 0)),
            pl.BlockSpec((_H, _U), lambda i: (0, 0)),
        ],
        out_shape=[
            jax.ShapeDtypeStruct((_H, _U, _PD), jnp.float32),
            jax.ShapeDtypeStruct((_H, _U), jnp.int32),
        ],
        scratch_shapes=[pltpu.VMEM((_H, _N_TOP, _L), jnp.float32)],
    )(m, q, k0, v64)
    out = _sc_scatter(updfix, gidx, jnp.zeros((_ZR, _PD), jnp.float32))
    out = out.reshape(_H, _L, _PD)[:, :, :_D]
    return jnp.swapaxes(out, 0, 1)[None]
